# baseline (speedup 1.0000x reference)
import jax
import jax.numpy as jnp
from jax.experimental import pallas as pl
from jax.experimental.pallas import tpu as pltpu

L = 4
D = 768
H = 12
HD = 64
P = 256
G = 16
K = 8
PS = 14
TEMP = 0.1
MLP_D = 3072
SCALE = HD ** -0.5
NEG = -1e9
B = 4
CIN = 3 * PS * PS


def _ln_rows(x, g, b):
    m = jnp.mean(x, axis=-1, keepdims=True)
    v = jnp.mean((x - m) ** 2, axis=-1, keepdims=True)
    return (x - m) * jax.lax.rsqrt(v + 1e-5) * g + b


def _l2n(x):
    n = jnp.sqrt(jnp.sum(x * x, axis=-1, keepdims=True))
    return x / jnp.maximum(n, 1e-12)


def _mm(a, b):
    return jnp.dot(a, b, preferred_element_type=jnp.float32)


def _mmT(a, b):
    return jax.lax.dot_general(a, b, (((1,), (1,)), ((), ())),
                               preferred_element_type=jnp.float32)


def _bf(x):
    return x.astype(jnp.bfloat16)


def _mmb(a, b):
    return jnp.dot(_bf(a), _bf(b), preferred_element_type=jnp.float32)


def _mmTb(a, b):
    return jax.lax.dot_general(_bf(a), _bf(b), (((1,), (1,)), ((), ())),
                               preferred_element_type=jnp.float32)


def _backbone_kernel(x2d_ref, pw_ref, pb_ref, pos_ref, tokc0_ref,
                     wq_h, wk_h, qkvw_h, projw_h, w1_h, w2_h,
                     bq_ref, bk_ref, bias_ref, qkvb_ref, projb_ref,
                     g1_ref, be1_ref, g2_ref, be2_ref, mb1_ref, mb2_ref,
                     lnfg_ref, lnfb_ref,
                     out_ref,
                     tokp_s, tokc_s, awq, awk, aqkv, aproj, m1_s, m2_s, sem):
    l = pl.program_id(0)

    def cp(hsrc, idx, dst, s):
        return pltpu.make_async_copy(hsrc.at[idx], dst, sem.at[s])

    @pl.when(l == 0)
    def _prologue():
        cp(wq_h, 0, awq, 0).start()
        cp(wk_h, 0, awk, 1).start()
        cp(qkvw_h, 0, aqkv, 2).start()
        cp(projw_h, 0, aproj, 3).start()
        z = _mmb(x2d_ref[...], pw_ref[...]) + pb_ref[...]
        pos = pos_ref[...]
        for b in range(B):
            tokp_s[b] = z[b * P:(b + 1) * P, :] + pos
        tokc_s[...] = tokc0_ref[...]

    cp(w1_h, l, m1_s, 4).start()
    cp(w2_h, l, m2_s, 5).start()

    tokp = tokp_s[...].reshape(B * P, D)
    tokc = tokc_s[...].reshape(B, D)
    g1 = g1_ref[0]
    be1 = be1_ref[0]
    xn_p = _ln_rows(tokp, g1, be1)
    xn_c = _ln_rows(tokc, g1, be1)

    cp(wq_h, l, awq, 0).wait()
    cp(wk_h, l, awk, 1).wait()

    q2 = _l2n(_mm(xn_p, awq[...]) + bq_ref[0])
    k2 = _l2n(_mm(xn_p, awk[...]) + bk_ref[0])

    cp(qkvw_h, l, aqkv, 2).wait()

    qkvb = qkvb_ref[0]
    qkv_p = _bf(_mmb(xn_p, aqkv[...]) + qkvb)
    qkv_c = _mmb(xn_c, aqkv[...]) + qkvb

    iota_q = jax.lax.broadcasted_iota(jnp.int32, (P, P), 1)
    iota_p = jax.lax.broadcasted_iota(jnp.int32, (P, P), 0)
    diag = iota_q == iota_p
    bias = bias_ref[0]

    op_rows = []
    oc_rows = []
    for b in range(B):
        r0 = b * P
        sc = _mmT(q2[r0:r0 + P, :], k2[r0:r0 + P, :]) + bias
        work = jnp.where(diag, NEG, sc)
        Lr = jnp.full((P, P), NEG, jnp.float32)
        den = jnp.zeros((P, 1), jnp.float32)
        for _ in range(K):
            m = jnp.max(work, axis=-1, keepdims=True)
            oh = work == m
            Lr = jnp.where(oh, m / TEMP, Lr)
            work = jnp.where(oh, NEG, work)
            den = den + jnp.exp(m / TEMP)
        routed = Lr > 0.5 * NEG
        IR = jnp.where(routed, 1.0 + 1e-6 * den * jnp.exp(-Lr), 1.0)

        oc_parts = []
        op_parts = []
        for h in range(H):
            q0 = h * HD
            Qh = qkv_p[r0:r0 + P, q0:q0 + HD]
            Kh = qkv_p[r0:r0 + P, D + q0:D + q0 + HD]
            Vh = qkv_p[r0:r0 + P, 2 * D + q0:2 * D + q0 + HD]
            qc = qkv_c[b:b + 1, q0:q0 + HD]
            kc = qkv_c[b:b + 1, D + q0:D + q0 + HD]
            vc = qkv_c[b:b + 1, 2 * D + q0:2 * D + q0 + HD]

            lp = _mmTb(qc, Kh) * SCALE
            ls = jnp.sum(qc * kc, axis=-1, keepdims=True) * SCALE
            mx = jnp.maximum(jnp.max(lp, axis=-1, keepdims=True), ls)
            ep = jnp.exp(lp - mx)
            ec = jnp.exp(ls - mx)
            denom_c = ec + jnp.sum(ep, axis=-1, keepdims=True)
            oc_parts.append((ec * vc + _mmb(ep, Vh)) / denom_c)

            X = _mmTb(Qh, Kh) * SCALE + Lr
            xm = jnp.max(X, axis=-1, keepdims=True)
            Ef = jnp.exp(X - xm)
            denom = jnp.sum(Ef * IR, axis=-1, keepdims=True)
            W = _bf(Ef * (1.0 / denom))
            op_parts.append(_mmb(W, Vh))

        op_rows.append(_bf(jnp.concatenate(op_parts, axis=-1)))
        oc_rows.append(jnp.concatenate(oc_parts, axis=-1))

    op2 = jnp.concatenate(op_rows, axis=0)
    oc2 = jnp.concatenate(oc_rows, axis=0)

    cp(projw_h, l, aproj, 3).wait()
    projw = aproj[...]
    projb = projb_ref[0]
    tokp1 = tokp + _mmb(op2, projw) + projb
    tokc1 = tokc + _mmb(oc2, projw) + projb

    @pl.when(l < L - 1)
    def _prefetch():
        cp(wq_h, l + 1, awq, 0).start()
        cp(wk_h, l + 1, awk, 1).start()
        cp(qkvw_h, l + 1, aqkv, 2).start()
        cp(projw_h, l + 1, aproj, 3).start()

    g2 = g2_ref[0]
    be2 = be2_ref[0]
    mb1 = mb1_ref[0]
    mb2 = mb2_ref[0]
    xn2_p = _ln_rows(tokp1, g2, be2)
    xn2_c = _ln_rows(tokc1, g2, be2)
    cp(w1_h, l, m1_s, 4).wait()
    w1 = m1_s[...]
    h_p = _bf(jax.nn.gelu(_mmb(xn2_p, w1) + mb1))
    h_c = jax.nn.gelu(_mmb(xn2_c, w1) + mb1)
    cp(w2_h, l, m2_s, 5).wait()
    w2 = m2_s[...]
    tokp2 = tokp1 + _mmb(h_p, w2) + mb2
    tokc2 = tokc1 + _mmb(h_c, w2) + mb2

    tokp_s[...] = tokp2.reshape(B, P, D)
    tokc_s[...] = tokc2.reshape(B, 1, D)

    @pl.when(l == L - 1)
    def _final():
        out_ref[...] = _ln_rows(tokc2, lnfg_ref[...], lnfb_ref[...])


def _const(shape):
    nd = len(shape)
    return pl.BlockSpec(shape, lambda l: (0,) * nd)


def _byl(shape):
    nd = len(shape)
    return pl.BlockSpec((1,) + shape, lambda l: (l,) + (0,) * nd)


def _any():
    return pl.BlockSpec(memory_space=pl.ANY)


def kernel(images, patch_w, patch_b, cls_token, pos_embed, router_wq, router_bq,
           router_wk, router_bk, pos_bias, qkv_w, qkv_b, proj_w, proj_b,
           ln1_g, ln1_b, ln2_g, ln2_b, mlp_w1, mlp_b1, mlp_w2, mlp_b2,
           lnf_g, lnf_b):
    x = images.reshape(B, 3, G, PS, G, PS).transpose(0, 2, 4, 1, 3, 5)
    x2d = x.reshape(B * P, CIN)
    pos_p = pos_embed[0, 1:, :]
    tok_c0 = jnp.broadcast_to(cls_token[0] + pos_embed[0, :1, :], (B, 1, D))

    out = pl.pallas_call(
        _backbone_kernel,
        grid=(L,),
        in_specs=[
            _const((B * P, CIN)),
            _const((CIN, D)),
            _const((1, D)),
            _const((P, D)),
            _const((B, 1, D)),
            _any(), _any(), _any(), _any(), _any(), _any(),
            _byl((1, D)), _byl((1, D)),
            _byl((P, P)),
            _byl((1, 3 * D)),
            _byl((1, D)),
            _byl((1, D)), _byl((1, D)),
            _byl((1, D)), _byl((1, D)),
            _byl((1, MLP_D)), _byl((1, D)),
            _const((1, D)), _const((1, D)),
        ],
        out_specs=pl.BlockSpec((B, D), lambda l: (0, 0)),
        out_shape=jax.ShapeDtypeStruct((B, D), jnp.float32),
        scratch_shapes=[
            pltpu.VMEM((B, P, D), jnp.float32),
            pltpu.VMEM((B, 1, D), jnp.float32),
            pltpu.VMEM((D, D), jnp.float32),
            pltpu.VMEM((D, D), jnp.float32),
            pltpu.VMEM((D, 3 * D), jnp.float32),
            pltpu.VMEM((D, D), jnp.float32),
            pltpu.VMEM((D, MLP_D), jnp.float32),
            pltpu.VMEM((MLP_D, D), jnp.float32),
            pltpu.SemaphoreType.DMA((6,)),
        ],
        compiler_params=pltpu.CompilerParams(
            dimension_semantics=("arbitrary",),
        ),
    )(
        x2d, patch_w, patch_b.reshape(1, D), pos_p, tok_c0,
        router_wq, router_wk, qkv_w, proj_w, mlp_w1, mlp_w2,
        router_bq.reshape(L, 1, D), router_bk.reshape(L, 1, D),
        pos_bias,
        qkv_b.reshape(L, 1, 3 * D),
        proj_b.reshape(L, 1, D),
        ln1_g.reshape(L, 1, D), ln1_b.reshape(L, 1, D),
        ln2_g.reshape(L, 1, D), ln2_b.reshape(L, 1, D),
        mlp_b1.reshape(L, 1, MLP_D), mlp_b2.reshape(L, 1, D),
        lnf_g.reshape(1, D), lnf_b.reshape(1, D),
    )
    return out


# speedup vs baseline: 1.0026x; 1.0026x over previous
import jax
import jax.numpy as jnp
from jax.experimental import pallas as pl
from jax.experimental.pallas import tpu as pltpu

L = 4
D = 768
H = 12
HD = 64
P = 256
G = 16
K = 8
PS = 14
TEMP = 0.1
MLP_D = 3072
SCALE = HD ** -0.5
NEG = -1e9
B = 4
CIN = 3 * PS * PS


def _ln_rows(x, g, b):
    m = jnp.mean(x, axis=-1, keepdims=True)
    v = jnp.mean((x - m) ** 2, axis=-1, keepdims=True)
    return (x - m) * jax.lax.rsqrt(v + 1e-5) * g + b


def _l2n(x):
    n = jnp.sqrt(jnp.sum(x * x, axis=-1, keepdims=True))
    return x / jnp.maximum(n, 1e-12)


def _mm(a, b):
    return jnp.dot(a, b, preferred_element_type=jnp.float32)


def _mmT(a, b):
    return jax.lax.dot_general(a, b, (((1,), (1,)), ((), ())),
                               preferred_element_type=jnp.float32)


def _bf(x):
    return x.astype(jnp.bfloat16)


def _mmb(a, b):
    return jnp.dot(_bf(a), _bf(b), preferred_element_type=jnp.float32)


def _mmTb(a, b):
    return jax.lax.dot_general(_bf(a), _bf(b), (((1,), (1,)), ((), ())),
                               preferred_element_type=jnp.float32)


def _backbone_kernel(x2d_ref, pw_ref, pb_ref, pos_ref, tokc0_ref,
                     wq_h, wk_h, qkvw_h, projw_h, w1_h, w2_h,
                     bq_ref, bk_ref, bias_ref, qkvb_ref, projb_ref,
                     g1_ref, be1_ref, g2_ref, be2_ref, mb1_ref, mb2_ref,
                     lnfg_ref, lnfb_ref,
                     out_ref,
                     tokp_s, tokc_s, awq, awk, aqkv, aproj, m1_s, m2_s, sem):
    l = pl.program_id(0)

    def cp(hsrc, idx, dst, s):
        return pltpu.make_async_copy(hsrc.at[idx], dst, sem.at[s])

    @pl.when(l == 0)
    def _prologue():
        cp(wq_h, 0, awq, 0).start()
        cp(wk_h, 0, awk, 1).start()
        cp(qkvw_h, 0, aqkv, 2).start()
        cp(projw_h, 0, aproj, 3).start()
        z = _mmb(x2d_ref[...], pw_ref[...]) + pb_ref[...]
        pos = pos_ref[...]
        for b in range(B):
            tokp_s[b] = z[b * P:(b + 1) * P, :] + pos
        tokc_s[...] = tokc0_ref[...]

    cp(w1_h, l, m1_s, 4).start()
    cp(w2_h, l, m2_s, 5).start()

    tokp = tokp_s[...].reshape(B * P, D)
    tokc = tokc_s[...].reshape(B, D)
    g1 = g1_ref[0]
    be1 = be1_ref[0]
    xn_p = _ln_rows(tokp, g1, be1)
    xn_c = _ln_rows(tokc, g1, be1)

    cp(wq_h, l, awq, 0).wait()
    cp(wk_h, l, awk, 1).wait()

    q2 = _l2n(_mm(xn_p, awq[...]) + bq_ref[0])
    k2 = _l2n(_mm(xn_p, awk[...]) + bk_ref[0])

    cp(qkvw_h, l, aqkv, 2).wait()

    qkvb = qkvb_ref[0]
    qkv_p = _bf(_mmb(xn_p, aqkv[...]) + qkvb)
    qkv_c = _mmb(xn_c, aqkv[...]) + qkvb

    @pl.when(l < L - 1)
    def _prefetch_attn():
        cp(wq_h, l + 1, awq, 0).start()
        cp(wk_h, l + 1, awk, 1).start()
        cp(qkvw_h, l + 1, aqkv, 2).start()

    iota_q = jax.lax.broadcasted_iota(jnp.int32, (P, P), 1)
    iota_p = jax.lax.broadcasted_iota(jnp.int32, (P, P), 0)
    diag = iota_q == iota_p
    bias = bias_ref[0]

    op_rows = []
    oc_rows = []
    for b in range(B):
        r0 = b * P
        sc = _mmT(q2[r0:r0 + P, :], k2[r0:r0 + P, :]) + bias
        work = jnp.where(diag, NEG, sc)
        Lr = jnp.full((P, P), NEG, jnp.float32)
        den = jnp.zeros((P, 1), jnp.float32)
        for _ in range(K):
            m = jnp.max(work, axis=-1, keepdims=True)
            oh = work == m
            Lr = jnp.where(oh, m / TEMP, Lr)
            work = jnp.where(oh, NEG, work)
            den = den + jnp.exp(m / TEMP)
        routed = Lr > 0.5 * NEG
        IR = jnp.where(routed, 1.0 + 1e-6 * den * jnp.exp(-Lr), 1.0)

        oc_parts = []
        op_parts = []
        for h in range(H):
            q0 = h * HD
            Qh = qkv_p[r0:r0 + P, q0:q0 + HD]
            Kh = qkv_p[r0:r0 + P, D + q0:D + q0 + HD]
            Vh = qkv_p[r0:r0 + P, 2 * D + q0:2 * D + q0 + HD]
            qc = qkv_c[b:b + 1, q0:q0 + HD]
            kc = qkv_c[b:b + 1, D + q0:D + q0 + HD]
            vc = qkv_c[b:b + 1, 2 * D + q0:2 * D + q0 + HD]

            lp = _mmTb(qc, Kh) * SCALE
            ls = jnp.sum(qc * kc, axis=-1, keepdims=True) * SCALE
            mx = jnp.maximum(jnp.max(lp, axis=-1, keepdims=True), ls)
            ep = jnp.exp(lp - mx)
            ec = jnp.exp(ls - mx)
            denom_c = ec + jnp.sum(ep, axis=-1, keepdims=True)
            oc_parts.append((ec * vc + _mmb(ep, Vh)) / denom_c)

            X = _mmTb(Qh, Kh) * SCALE + Lr
            xm = jnp.max(X, axis=-1, keepdims=True)
            Ef = jnp.exp(X - xm)
            denom = jnp.sum(Ef * IR, axis=-1, keepdims=True)
            W = _bf(Ef * (1.0 / denom))
            op_parts.append(_mmb(W, Vh))

        op_rows.append(_bf(jnp.concatenate(op_parts, axis=-1)))
        oc_rows.append(jnp.concatenate(oc_parts, axis=-1))

    op2 = jnp.concatenate(op_rows, axis=0)
    oc2 = jnp.concatenate(oc_rows, axis=0)

    cp(projw_h, l, aproj, 3).wait()
    projw = aproj[...]
    projb = projb_ref[0]
    tokp1 = tokp + _mmb(op2, projw) + projb
    tokc1 = tokc + _mmb(oc2, projw) + projb

    @pl.when(l < L - 1)
    def _prefetch_proj():
        cp(projw_h, l + 1, aproj, 3).start()

    g2 = g2_ref[0]
    be2 = be2_ref[0]
    mb1 = mb1_ref[0]
    mb2 = mb2_ref[0]
    xn2_p = _ln_rows(tokp1, g2, be2)
    xn2_c = _ln_rows(tokc1, g2, be2)
    cp(w1_h, l, m1_s, 4).wait()
    w1 = m1_s[...]
    h_p = _bf(jax.nn.gelu(_mmb(xn2_p, w1) + mb1))
    h_c = jax.nn.gelu(_mmb(xn2_c, w1) + mb1)
    cp(w2_h, l, m2_s, 5).wait()
    w2 = m2_s[...]
    tokp2 = tokp1 + _mmb(h_p, w2) + mb2
    tokc2 = tokc1 + _mmb(h_c, w2) + mb2

    tokp_s[...] = tokp2.reshape(B, P, D)
    tokc_s[...] = tokc2.reshape(B, 1, D)

    @pl.when(l == L - 1)
    def _final():
        out_ref[...] = _ln_rows(tokc2, lnfg_ref[...], lnfb_ref[...])


def _const(shape):
    nd = len(shape)
    return pl.BlockSpec(shape, lambda l: (0,) * nd)


def _byl(shape):
    nd = len(shape)
    return pl.BlockSpec((1,) + shape, lambda l: (l,) + (0,) * nd)


def _any():
    return pl.BlockSpec(memory_space=pl.ANY)


def kernel(images, patch_w, patch_b, cls_token, pos_embed, router_wq, router_bq,
           router_wk, router_bk, pos_bias, qkv_w, qkv_b, proj_w, proj_b,
           ln1_g, ln1_b, ln2_g, ln2_b, mlp_w1, mlp_b1, mlp_w2, mlp_b2,
           lnf_g, lnf_b):
    x = images.reshape(B, 3, G, PS, G, PS).transpose(0, 2, 4, 1, 3, 5)
    x2d = x.reshape(B * P, CIN)
    pos_p = pos_embed[0, 1:, :]
    tok_c0 = jnp.broadcast_to(cls_token[0] + pos_embed[0, :1, :], (B, 1, D))

    out = pl.pallas_call(
        _backbone_kernel,
        grid=(L,),
        in_specs=[
            _const((B * P, CIN)),
            _const((CIN, D)),
            _const((1, D)),
            _const((P, D)),
            _const((B, 1, D)),
            _any(), _any(), _any(), _any(), _any(), _any(),
            _byl((1, D)), _byl((1, D)),
            _byl((P, P)),
            _byl((1, 3 * D)),
            _byl((1, D)),
            _byl((1, D)), _byl((1, D)),
            _byl((1, D)), _byl((1, D)),
            _byl((1, MLP_D)), _byl((1, D)),
            _const((1, D)), _const((1, D)),
        ],
        out_specs=pl.BlockSpec((B, D), lambda l: (0, 0)),
        out_shape=jax.ShapeDtypeStruct((B, D), jnp.float32),
        scratch_shapes=[
            pltpu.VMEM((B, P, D), jnp.float32),
            pltpu.VMEM((B, 1, D), jnp.float32),
            pltpu.VMEM((D, D), jnp.float32),
            pltpu.VMEM((D, D), jnp.float32),
            pltpu.VMEM((D, 3 * D), jnp.float32),
            pltpu.VMEM((D, D), jnp.float32),
            pltpu.VMEM((D, MLP_D), jnp.float32),
            pltpu.VMEM((MLP_D, D), jnp.float32),
            pltpu.SemaphoreType.DMA((6,)),
        ],
        compiler_params=pltpu.CompilerParams(
            dimension_semantics=("arbitrary",),
        ),
    )(
        x2d, patch_w, patch_b.reshape(1, D), pos_p, tok_c0,
        router_wq, router_wk, qkv_w, proj_w, mlp_w1, mlp_w2,
        router_bq.reshape(L, 1, D), router_bk.reshape(L, 1, D),
        pos_bias,
        qkv_b.reshape(L, 1, 3 * D),
        proj_b.reshape(L, 1, D),
        ln1_g.reshape(L, 1, D), ln1_b.reshape(L, 1, D),
        ln2_g.reshape(L, 1, D), ln2_b.reshape(L, 1, D),
        mlp_b1.reshape(L, 1, MLP_D), mlp_b2.reshape(L, 1, D),
        lnf_g.reshape(1, D), lnf_b.reshape(1, D),
    )
    return out
